# baseline (speedup 1.0000x reference)
import functools

import jax
import jax.numpy as jnp
from jax.experimental import pallas as pl
from jax.experimental.pallas import tpu as pltpu

EMA_EPS = 1e-12


def _bmm(a, b):
    return jax.lax.dot_general(
        a, b,
        dimension_numbers=(((2,), (1,)), ((0,), (0,))),
        preferred_element_type=jnp.float32,
    )


def _ema_kernel(pt_ref, z_ref, out_ref, zb_ref, ob_ref, ld_sem, st_sem, *,
                T, K):
    B = pt_ref.shape[0]

    loads = []
    for k in range(K):
        ld = pltpu.make_async_copy(
            z_ref.at[:, pl.ds(k * T, T), :], zb_ref.at[k], ld_sem.at[k])
        ld.start()
        loads.append(ld)

    jj = jax.lax.broadcasted_iota(jnp.int32, (T, T), 0)
    ii = jax.lax.broadcasted_iota(jnp.int32, (T, T), 1)
    cum_mat = jnp.where(jj <= ii, 1.0, 0.0)
    tril = (jj >= ii)[None]

    def build(p):
        logd = jnp.log(jnp.maximum(1.0 - p, EMA_EPS))
        S = jnp.dot(logd, cum_mat, preferred_element_type=jnp.float32)
        Sp = S - jnp.log(p)
        delta = S[:, :, None] - Sp[:, None, :]
        delta = jnp.where(tril, delta, -jnp.inf)
        return jnp.exp(delta), jnp.exp(S)

    Ws = [build(pt_ref[:, 0, k * T:(k + 1) * T]) for k in range(K)]

    state = None
    stores = []
    for k in range(K):
        W, cw = Ws[k]
        loads[k].wait()
        res = _bmm(W, zb_ref[k])
        if state is not None:
            res = res + cw[:, :, None] * state[:, None, :]
        ob_ref[k] = res
        st = pltpu.make_async_copy(
            ob_ref.at[k], out_ref.at[:, pl.ds(k * T, T), :], st_sem.at[k])
        st.start()
        stores.append(st)
        state = res[:, T - 1, :]

    for st in stores:
        st.wait()


@jax.jit
def kernel(z, pt):
    B, L, D = z.shape
    K = 16
    T = L // K

    body = functools.partial(_ema_kernel, T=T, K=K)
    return pl.pallas_call(
        body,
        grid=(1,),
        in_specs=[
            pl.BlockSpec((B, 1, L), lambda i: (0, 0, 0)),
            pl.BlockSpec(memory_space=pl.ANY),
        ],
        out_specs=pl.BlockSpec(memory_space=pl.ANY),
        out_shape=jax.ShapeDtypeStruct((B, L, D), jnp.float32),
        scratch_shapes=[
            pltpu.VMEM((K, B, T, D), jnp.float32),
            pltpu.VMEM((K, B, T, D), jnp.float32),
            pltpu.SemaphoreType.DMA((K,)),
            pltpu.SemaphoreType.DMA((K,)),
        ],
    )(pt.reshape(B, 1, L), z)


# speedup vs baseline: 1.1676x; 1.1676x over previous
import functools

import jax
import jax.numpy as jnp
from jax.experimental import pallas as pl
from jax.experimental.pallas import tpu as pltpu

EMA_EPS = 1e-12


def _bmm(a, b):
    return jax.lax.dot_general(
        a, b,
        dimension_numbers=(((2,), (1,)), ((0,), (0,))),
        preferred_element_type=jnp.float32,
    )


def _ema_kernel(pt_ref, z_ref, out_ref, zb_ref, ob_ref, ld_sem, st_sem, *,
                T, K):
    B = pt_ref.shape[0]

    H = B // 2
    loads = []
    for k in range(K):
        lda = pltpu.make_async_copy(
            z_ref.at[pl.ds(0, H), pl.ds(k * T, T), :],
            zb_ref.at[k, pl.ds(0, H)], ld_sem.at[k, 0])
        ldb = pltpu.make_async_copy(
            z_ref.at[pl.ds(H, H), pl.ds(k * T, T), :],
            zb_ref.at[k, pl.ds(H, H)], ld_sem.at[k, 1])
        lda.start()
        ldb.start()
        loads.append((lda, ldb))

    jj = jax.lax.broadcasted_iota(jnp.int32, (T, T), 0)
    ii = jax.lax.broadcasted_iota(jnp.int32, (T, T), 1)
    cum_mat = jnp.where(jj <= ii, 1.0, 0.0)
    tril = (jj >= ii)[None]

    def build(p):
        logd = jnp.log(jnp.maximum(1.0 - p, EMA_EPS))
        S = jnp.dot(logd, cum_mat, preferred_element_type=jnp.float32)
        Sp = S - jnp.log(p)
        delta = S[:, :, None] - Sp[:, None, :]
        delta = jnp.where(tril, delta, -jnp.inf)
        return jnp.exp(delta), jnp.exp(S)

    Ws = [build(pt_ref[:, 0, k * T:(k + 1) * T]) for k in range(K)]

    state = None
    stores = []
    for k in range(K):
        W, cw = Ws[k]
        loads[k][0].wait()
        loads[k][1].wait()
        res = _bmm(W, zb_ref[k])
        if state is not None:
            res = res + cw[:, :, None] * state[:, None, :]
        ob_ref[k] = res
        st = pltpu.make_async_copy(
            ob_ref.at[k], out_ref.at[:, pl.ds(k * T, T), :], st_sem.at[k])
        st.start()
        stores.append(st)
        state = res[:, T - 1, :]

    for st in stores:
        st.wait()


@jax.jit
def kernel(z, pt):
    B, L, D = z.shape
    K = 8
    T = L // K

    body = functools.partial(_ema_kernel, T=T, K=K)
    return pl.pallas_call(
        body,
        grid=(1,),
        in_specs=[
            pl.BlockSpec((B, 1, L), lambda i: (0, 0, 0)),
            pl.BlockSpec(memory_space=pl.ANY),
        ],
        out_specs=pl.BlockSpec(memory_space=pl.ANY),
        out_shape=jax.ShapeDtypeStruct((B, L, D), jnp.float32),
        scratch_shapes=[
            pltpu.VMEM((K, B, T, D), jnp.float32),
            pltpu.VMEM((K, B, T, D), jnp.float32),
            pltpu.SemaphoreType.DMA((K, 2)),
            pltpu.SemaphoreType.DMA((K,)),
        ],
    )(pt.reshape(B, 1, L), z)


# speedup vs baseline: 1.2115x; 1.0376x over previous
import functools

import jax
import jax.numpy as jnp
from jax.experimental import pallas as pl
from jax.experimental.pallas import tpu as pltpu

EMA_EPS = 1e-12


def _bmm(a, b):
    return jax.lax.dot_general(
        a, b,
        dimension_numbers=(((2,), (1,)), ((0,), (0,))),
        preferred_element_type=jnp.float32,
    )


def _ema_kernel(pt_ref, z_ref, out_ref, zb_ref, ob_ref, ld_sem, st_sem, *,
                T, K):
    B = pt_ref.shape[0]
    D = z_ref.shape[2]

    loads = []
    for k in range(K):
        ld = pltpu.make_async_copy(
            z_ref.at[:, pl.ds(k * T, T), :],
            zb_ref.at[k, :, pl.ds(0, T), :], ld_sem.at[k])
        ld.start()
        loads.append(ld)

    zb_ref[:, :, T:, :] = jnp.zeros((K, B, 8, D), jnp.float32)

    rr = jax.lax.broadcasted_iota(jnp.int32, (T, T + 8), 0)
    cc = jax.lax.broadcasted_iota(jnp.int32, (T, T + 8), 1)
    jjT = jax.lax.broadcasted_iota(jnp.int32, (T, T), 0)
    iiT = jax.lax.broadcasted_iota(jnp.int32, (T, T), 1)
    cum_mat = jnp.where(jjT <= iiT, 1.0, 0.0)
    mask = jnp.logical_or(cc <= rr, cc == T)[None]
    pad8 = jnp.where(
        jax.lax.broadcasted_iota(jnp.int32, (1, 8), 1) == 0,
        0.0, jnp.inf)

    def build(p):
        logd = jnp.log(jnp.maximum(1.0 - p, EMA_EPS))
        S = jnp.dot(logd, cum_mat, preferred_element_type=jnp.float32)
        Sp = S - jnp.log(p)
        Sp_aug = jnp.concatenate(
            [Sp, jnp.broadcast_to(pad8, (p.shape[0], 8))], axis=1)
        delta = S[:, :, None] - Sp_aug[:, None, :]
        delta = jnp.where(mask, delta, -jnp.inf)
        return jnp.exp(delta)

    Ws = [build(pt_ref[:, 0, k * T:(k + 1) * T]) for k in range(K)]

    stores = []
    for k in range(K):
        loads[k].wait()
        res = _bmm(Ws[k], zb_ref[k])
        ob_ref[k] = res
        st = pltpu.make_async_copy(
            ob_ref.at[k], out_ref.at[:, pl.ds(k * T, T), :], st_sem.at[k])
        st.start()
        stores.append(st)
        if k + 1 < K:
            zb_ref[k + 1, :, T, :] = res[:, T - 1, :]

    for st in stores:
        st.wait()


@jax.jit
def kernel(z, pt):
    B, L, D = z.shape
    K = 8
    T = L // K

    body = functools.partial(_ema_kernel, T=T, K=K)
    return pl.pallas_call(
        body,
        grid=(1,),
        in_specs=[
            pl.BlockSpec((B, 1, L), lambda i: (0, 0, 0)),
            pl.BlockSpec(memory_space=pl.ANY),
        ],
        out_specs=pl.BlockSpec(memory_space=pl.ANY),
        out_shape=jax.ShapeDtypeStruct((B, L, D), jnp.float32),
        scratch_shapes=[
            pltpu.VMEM((K, B, T + 8, D), jnp.float32),
            pltpu.VMEM((K, B, T, D), jnp.float32),
            pltpu.SemaphoreType.DMA((K,)),
            pltpu.SemaphoreType.DMA((K,)),
        ],
    )(pt.reshape(B, 1, L), z)
